# Initial kernel scaffold; baseline (speedup 1.0000x reference)
import jax
import jax.numpy as jnp
from jax.experimental import pallas as pl


def kernel(sequence, token_table, pos_table):
    raise NotImplementedError("write your pallas kernel here")


# speedup vs baseline: 2.6011x; 2.6011x over previous
import functools

import jax
import jax.numpy as jnp
from jax import lax
from jax.experimental import pallas as pl
from jax.experimental.pallas import tpu as pltpu
from jax.experimental.pallas import tpu_sc as plsc

VOCAB = 100000
D = 64
S = 200
B = 1024
NW = 32
SEQ_PER_W = B // NW
LANES = 16


def _make_kernel():
    mesh = plsc.VectorSubcoreMesh(core_axis_name="c", subcore_axis_name="s")

    @functools.partial(
        pl.kernel,
        mesh=mesh,
        out_type=jax.ShapeDtypeStruct((B * S, D), jnp.float32),
        scratch_types=[
            pltpu.VMEM((S,), jnp.int32),
            pltpu.VMEM((S, D), jnp.float32),
            pltpu.VMEM((S, D), jnp.float32),
            pltpu.SemaphoreType.DMA,
        ],
        compiler_params=pltpu.CompilerParams(use_tc_tiling_on_sc=False),
    )
    def k(seq_hbm, tok_hbm, pos_hbm, out_hbm, idx_v, rows_v, pos_v, sem):
        wid = lax.axis_index("s") * 2 + lax.axis_index("c")

        pltpu.sync_copy(pos_hbm, pos_v)

        def add_body(r, carry):
            for q in range(D // LANES):
                plsc.addupdate(
                    rows_v.at[r, pl.ds(q * LANES, LANES)],
                    pos_v[r, pl.ds(q * LANES, LANES)],
                )
            return carry

        def seq_body(sidx, carry):
            row = (wid * SEQ_PER_W + sidx) * S
            pltpu.sync_copy(seq_hbm.at[pl.ds(row, S)], idx_v)
            cp1 = pltpu.async_copy(
                tok_hbm.at[idx_v.at[pl.ds(0, 128)]], rows_v.at[pl.ds(0, 128)], sem)
            cp2 = pltpu.async_copy(
                tok_hbm.at[idx_v.at[pl.ds(128, S - 128)]],
                rows_v.at[pl.ds(128, S - 128)], sem)
            cp1.wait()
            cp2.wait()
            lax.fori_loop(0, S, add_body, 0)
            pltpu.sync_copy(rows_v, out_hbm.at[pl.ds(row, S)])
            return carry

        lax.fori_loop(0, SEQ_PER_W, seq_body, 0)

    return k


_kernel_call = _make_kernel()


def kernel(sequence, token_table, pos_table):
    seq_flat = sequence.reshape(-1).astype(jnp.int32)
    out = _kernel_call(seq_flat, token_table, pos_table)
    return out.reshape(B, S, D)


# speedup vs baseline: 3.2529x; 1.2506x over previous
import functools

import jax
import jax.numpy as jnp
from jax import lax
from jax.experimental import pallas as pl
from jax.experimental.pallas import tpu as pltpu
from jax.experimental.pallas import tpu_sc as plsc

VOCAB = 100000
D = 64
S = 200
B = 1024
NW = 32
SEQ_PER_W = B // NW
LANES = 16
NB = 4
SPLIT = 128


def _make_kernel():
    mesh = plsc.VectorSubcoreMesh(core_axis_name="c", subcore_axis_name="s")

    @functools.partial(
        pl.kernel,
        mesh=mesh,
        out_type=jax.ShapeDtypeStruct((B * S, D), jnp.float32),
        scratch_types=[
            pltpu.VMEM((SEQ_PER_W * S,), jnp.int32),
            pltpu.VMEM((NB, S, D), jnp.float32),
            pltpu.VMEM((S, D), jnp.float32),
            pltpu.SemaphoreType.DMA((NB,)),
            pltpu.SemaphoreType.DMA((NB,)),
        ],
        compiler_params=pltpu.CompilerParams(use_tc_tiling_on_sc=False),
    )
    def k(seq_hbm, tok_hbm, pos_hbm, out_hbm, idx_v, rows_v, pos_v, gsem, osem):
        wid = lax.axis_index("s") * 2 + lax.axis_index("c")
        base_row = wid * (SEQ_PER_W * S)

        pltpu.sync_copy(seq_hbm.at[pl.ds(base_row, SEQ_PER_W * S)], idx_v)
        pltpu.sync_copy(pos_hbm, pos_v)

        def gather_descs(s, b):
            off = s * S
            c1 = pltpu.make_async_copy(
                tok_hbm.at[idx_v.at[pl.ds(off, SPLIT)]],
                rows_v.at[b, pl.ds(0, SPLIT)], gsem.at[b])
            c2 = pltpu.make_async_copy(
                tok_hbm.at[idx_v.at[pl.ds(off + SPLIT, S - SPLIT)]],
                rows_v.at[b, pl.ds(SPLIT, S - SPLIT)], gsem.at[b])
            return c1, c2

        def out_desc(s, b):
            return pltpu.make_async_copy(
                rows_v.at[b], out_hbm.at[pl.ds(base_row + s * S, S)], osem.at[b])

        def add_slot(b):
            def add_body(i, c):
                r = i * 4
                for j in range(4):
                    for q in range(D // LANES):
                        plsc.addupdate(
                            rows_v.at[b, r + j, pl.ds(q * LANES, LANES)],
                            pos_v[r + j, pl.ds(q * LANES, LANES)],
                        )
                return c
            lax.fori_loop(0, S // 4, add_body, 0)

        for s0 in range(NB - 1):
            c1, c2 = gather_descs(s0, s0)
            c1.start()
            c2.start()

        def body(i, carry):
            s_base = i * NB
            for b in range(NB):
                s = s_base + b
                c1, c2 = gather_descs(s, b)
                c1.wait()
                c2.wait()
                add_slot(b)
                out_desc(s, b).start()
                t = s + NB - 1
                bt = (b + NB - 1) % NB

                @pl.when(t < SEQ_PER_W)
                def _prefetch():
                    @pl.when(s >= 1)
                    def _drain_prev():
                        out_desc(s - 1, bt).wait()
                    g1, g2 = gather_descs(t, bt)
                    g1.start()
                    g2.start()
            return carry

        lax.fori_loop(0, SEQ_PER_W // NB, body, 0)

        for b in range(NB):
            out_desc(SEQ_PER_W - NB + b, b).wait()

    return k


_kernel_call = _make_kernel()


def kernel(sequence, token_table, pos_table):
    seq_flat = sequence.reshape(-1).astype(jnp.int32)
    out = _kernel_call(seq_flat, token_table, pos_table)
    return out.reshape(B, S, D)


# speedup vs baseline: 4.7340x; 1.4553x over previous
import functools

import jax
import jax.numpy as jnp
from jax import lax
from jax.experimental import pallas as pl
from jax.experimental.pallas import tpu as pltpu
from jax.experimental.pallas import tpu_sc as plsc

VOCAB = 100000
D = 64
DP = 128
S = 200
B = 1024
NW = 32
SEQ_PER_W = B // NW
LANES = 16
NB = 4
SPLIT = 128


def _make_kernel():
    mesh = plsc.VectorSubcoreMesh(core_axis_name="c", subcore_axis_name="s")

    @functools.partial(
        pl.kernel,
        mesh=mesh,
        out_type=jax.ShapeDtypeStruct((B * S, DP), jnp.float32),
        scratch_types=[
            pltpu.VMEM((SEQ_PER_W * S,), jnp.int32),
            pltpu.VMEM((NB, S, D), jnp.float32),
            pltpu.VMEM((S * D,), jnp.float32),
            pltpu.SemaphoreType.DMA((NB,)),
            pltpu.SemaphoreType.DMA((NB,)),
        ],
        compiler_params=pltpu.CompilerParams(use_tc_tiling_on_sc=False),
    )
    def k(seq_hbm, tok_hbm, pos_hbm, out_hbm, idx_v, rows_v, pos_v, gsem, osem):
        wid = lax.axis_index("s") * 2 + lax.axis_index("c")
        base_row = wid * (SEQ_PER_W * S)

        pltpu.sync_copy(seq_hbm.at[pl.ds(base_row, SEQ_PER_W * S)], idx_v)
        pltpu.sync_copy(pos_hbm, pos_v)

        def gather_descs(s, b):
            off = s * S
            c1 = pltpu.make_async_copy(
                tok_hbm.at[idx_v.at[pl.ds(off, SPLIT)]],
                rows_v.at[b, pl.ds(0, SPLIT)], gsem.at[b])
            c2 = pltpu.make_async_copy(
                tok_hbm.at[idx_v.at[pl.ds(off + SPLIT, S - SPLIT)]],
                rows_v.at[b, pl.ds(SPLIT, S - SPLIT)], gsem.at[b])
            return c1, c2

        def out_desc(s, b):
            return pltpu.make_async_copy(
                rows_v.at[b],
                out_hbm.at[pl.ds(base_row + s * S, S), pl.ds(0, D)],
                osem.at[b])

        def add_slot(b):
            def add_body(i, c):
                r = i * 4
                for j in range(4):
                    for q in range(D // LANES):
                        plsc.addupdate(
                            rows_v.at[b, r + j, pl.ds(q * LANES, LANES)],
                            pos_v[pl.ds((r + j) * D + q * LANES, LANES)],
                        )
                return c
            lax.fori_loop(0, S // 4, add_body, 0)

        for s0 in range(NB - 1):
            c1, c2 = gather_descs(s0, s0)
            c1.start()
            c2.start()

        def body(i, carry):
            s_base = i * NB
            for b in range(NB):
                s = s_base + b
                c1, c2 = gather_descs(s, b)
                c1.wait()
                c2.wait()
                add_slot(b)
                out_desc(s, b).start()
                t = s + NB - 1
                bt = (b + NB - 1) % NB

                @pl.when(t < SEQ_PER_W)
                def _prefetch():
                    @pl.when(s >= 1)
                    def _drain_prev():
                        out_desc(s - 1, bt).wait()
                    g1, g2 = gather_descs(t, bt)
                    g1.start()
                    g2.start()
            return carry

        lax.fori_loop(0, SEQ_PER_W // NB, body, 0)

        for b in range(NB):
            out_desc(SEQ_PER_W - NB + b, b).wait()

    return k


_kernel_call = _make_kernel()


def kernel(sequence, token_table, pos_table):
    seq_flat = sequence.reshape(-1).astype(jnp.int32)
    pos_flat = pos_table.reshape(-1)
    out = _kernel_call(seq_flat, token_table, pos_flat)
    return out[:, :D].reshape(B, S, D)
